# baseline (speedup 1.0000x reference)
import jax
import jax.numpy as jnp
from jax.experimental import pallas as pl
from jax.experimental.pallas import tpu as pltpu

_BB = 8


def _pe_kernel(row_ref, col_ref, o_ref, scratch_ref):
    @pl.when(pl.program_id(0) == 0)
    def _build():
        col0 = col_ref[0:1, :]
        col1 = col_ref[1:2, :]
        row0 = row_ref[0:1, :]
        row1 = row_ref[1:2, :]
        mq = [
            jnp.concatenate([col0, row0], axis=1),
            jnp.concatenate([col1, row0], axis=1),
            jnp.concatenate([col0, row1], axis=1),
            jnp.concatenate([col1, row1], axis=1),
        ]
        n = 4 * mq[0].shape[1]
        rep = [jnp.repeat(m, 4, axis=1) for m in mq]
        k = jax.lax.broadcasted_iota(jnp.int32, (1, n), 1)
        qsel = k & 3
        row = jnp.where(
            qsel == 0,
            rep[0],
            jnp.where(qsel == 1, rep[1], jnp.where(qsel == 2, rep[2], rep[3])),
        )
        scratch_ref[...] = jnp.broadcast_to(row, scratch_ref.shape)

    o_ref[...] = scratch_ref[...]


def kernel(x, row_embed, col_embed):
    b, _, h, w = x.shape
    d = row_embed.shape[1]
    row_len = 2 * d * h * w
    out = pl.pallas_call(
        _pe_kernel,
        grid=(b // _BB,),
        in_specs=[
            pl.BlockSpec((2, d), lambda i: (0, 0)),
            pl.BlockSpec((2, d), lambda i: (0, 0)),
        ],
        out_specs=pl.BlockSpec((_BB, row_len), lambda i: (i, 0)),
        out_shape=jax.ShapeDtypeStruct((b, row_len), x.dtype),
        scratch_shapes=[pltpu.VMEM((_BB, row_len), jnp.float32)],
    )(row_embed, col_embed)
    return out.reshape(b, 2 * d, h, w)


# speedup vs baseline: 1.0820x; 1.0820x over previous
import jax
import jax.numpy as jnp
from jax.experimental import pallas as pl
from jax.experimental.pallas import tpu as pltpu

_BB = 16


def _pe_kernel(row_ref, col_ref, o_ref, scratch_ref, sem):
    col0 = col_ref[0:1, :]
    col1 = col_ref[1:2, :]
    row0 = row_ref[0:1, :]
    row1 = row_ref[1:2, :]
    mq = [
        jnp.concatenate([col0, row0], axis=1),
        jnp.concatenate([col1, row0], axis=1),
        jnp.concatenate([col0, row1], axis=1),
        jnp.concatenate([col1, row1], axis=1),
    ]
    n = 4 * mq[0].shape[1]
    rep = [jnp.repeat(m, 4, axis=1) for m in mq]
    k = jax.lax.broadcasted_iota(jnp.int32, (1, n), 1)
    qsel = k & 3
    row = jnp.where(
        qsel == 0,
        rep[0],
        jnp.where(qsel == 1, rep[1], jnp.where(qsel == 2, rep[2], rep[3])),
    )
    scratch_ref[...] = jnp.broadcast_to(row, scratch_ref.shape)

    b = o_ref.shape[0]
    copies = [
        pltpu.make_async_copy(
            scratch_ref, o_ref.at[pl.ds(t * _BB, _BB), :], sem
        )
        for t in range(b // _BB)
    ]
    for c in copies:
        c.start()
    for c in copies:
        c.wait()


def kernel(x, row_embed, col_embed):
    b, _, h, w = x.shape
    d = row_embed.shape[1]
    row_len = 2 * d * h * w
    out = pl.pallas_call(
        _pe_kernel,
        in_specs=[
            pl.BlockSpec(memory_space=pltpu.MemorySpace.VMEM),
            pl.BlockSpec(memory_space=pltpu.MemorySpace.VMEM),
        ],
        out_specs=pl.BlockSpec(memory_space=pl.ANY),
        out_shape=jax.ShapeDtypeStruct((b, row_len), x.dtype),
        scratch_shapes=[
            pltpu.VMEM((_BB, row_len), jnp.float32),
            pltpu.SemaphoreType.DMA,
        ],
    )(row_embed, col_embed)
    return out.reshape(b, 2 * d, h, w)


# speedup vs baseline: 3.1003x; 2.8652x over previous
import jax
import jax.numpy as jnp
from jax.experimental import pallas as pl
from jax.experimental.pallas import tpu as pltpu

_BB = 16


def _pe_kernel(row_ref, col_ref, o_ref, scratch_ref, sem):
    col0 = col_ref[0:1, :]
    col1 = col_ref[1:2, :]
    row0 = row_ref[0:1, :]
    row1 = row_ref[1:2, :]
    row = jnp.concatenate(
        [col0, row0, col1, row0, col0, row1, col1, row1], axis=1
    )
    scratch_ref[...] = jnp.broadcast_to(row, scratch_ref.shape)

    b = o_ref.shape[0]
    copies = [
        pltpu.make_async_copy(
            scratch_ref, o_ref.at[pl.ds(t * _BB, _BB), :], sem
        )
        for t in range(b // _BB)
    ]
    for c in copies:
        c.start()
    for c in copies:
        c.wait()


def kernel(x, row_embed, col_embed):
    b, _, h, w = x.shape
    d = row_embed.shape[1]
    row_len = 2 * d * h * w
    out = pl.pallas_call(
        _pe_kernel,
        in_specs=[
            pl.BlockSpec(memory_space=pltpu.MemorySpace.VMEM),
            pl.BlockSpec(memory_space=pltpu.MemorySpace.VMEM),
        ],
        out_specs=pl.BlockSpec(memory_space=pl.ANY),
        out_shape=jax.ShapeDtypeStruct((b, row_len), x.dtype),
        scratch_shapes=[
            pltpu.VMEM((_BB, row_len), jnp.float32),
            pltpu.SemaphoreType.DMA,
        ],
    )(row_embed, col_embed)
    return out.reshape(b, h, w, 2 * d).transpose(0, 3, 1, 2)
